# Initial kernel scaffold; baseline (speedup 1.0000x reference)
import jax
import jax.numpy as jnp
from jax.experimental import pallas as pl


def kernel(x, edge_index0, edge_index1, W0, b0, W1, b1):
    raise NotImplementedError("write your pallas kernel here")


# speedup vs baseline: 8.4013x; 8.4013x over previous
import functools

import jax
import jax.numpy as jnp
from jax import lax
from jax.experimental import pallas as pl
from jax.experimental.pallas import tpu as pltpu
from jax.experimental.pallas import tpu_sc as plsc

_N = 10000
_D_IN = 128
_D_H = 256
_N_CLS = 64
_E = 320000

_NC = 2
_NS = 16
_CH = 128
_SUB = 4
_EPT = 20480
_E_PAD = _NS * _EPT
_ITERS = _EPT // (_CH * _SUB)
_N_PAD = 10240
_PAD_ROWS = _N_PAD - _N
_OPT = 624


def _agg_body(table, src2d, dst2d, zrows, z16, ones16, out_sum, out_cnt,
              src_v, dst_v, rows_v, ones_v, acc, cnt, sem, *, d2):
    c = lax.axis_index("c")
    s = lax.axis_index("s")

    rpt = _N_PAD // _NS
    r0 = s * rpt
    pltpu.sync_copy(zrows.at[pl.ds(r0, rpt)], acc.at[pl.ds(r0, rpt)])
    pltpu.sync_copy(z16.at[pl.ds(r0, rpt)], cnt.at[pl.ds(r0, rpt)])
    pltpu.sync_copy(ones16, ones_v)
    plsc.subcore_barrier()

    row0 = s * (_EPT // _CH)
    tbl = table.at[c]

    def body(i, carry):
        rb = row0 + i * _SUB
        pltpu.sync_copy(src2d.at[pl.ds(rb, _SUB)], src_v)
        pltpu.sync_copy(dst2d.at[pl.ds(rb, _SUB)], dst_v)
        cps = [pltpu.async_copy(tbl.at[src_v.at[j]], rows_v.at[j], sem)
               for j in range(_SUB)]
        for cp in cps:
            cp.wait()
        for j in range(_SUB):
            pltpu.sync_copy(rows_v.at[j], acc.at[dst_v.at[j]], add=True)
            pltpu.sync_copy(ones_v, cnt.at[dst_v.at[j]], add=True)
        return carry

    lax.fori_loop(0, _ITERS, body, 0)
    plsc.subcore_barrier()

    o0 = s * _OPT
    rem0 = (_NS - 1) * _OPT
    rem = _N - rem0

    @pl.when(s < _NS - 1)
    def _():
        pltpu.sync_copy(acc.at[pl.ds(o0, _OPT)], out_sum.at[c, pl.ds(o0, _OPT)])
        pltpu.sync_copy(cnt.at[pl.ds(o0, _OPT)], out_cnt.at[c, pl.ds(o0, _OPT)])

    @pl.when(s == _NS - 1)
    def _():
        pltpu.sync_copy(acc.at[pl.ds(rem0, rem)], out_sum.at[c, pl.ds(rem0, rem)])
        pltpu.sync_copy(cnt.at[pl.ds(rem0, rem)], out_cnt.at[c, pl.ds(rem0, rem)])


def _make_agg(d2):
    mesh = plsc.VectorSubcoreMesh(core_axis_name="c", subcore_axis_name="s",
                                  num_cores=_NC, num_subcores=_NS)
    return functools.partial(
        pl.kernel,
        mesh=mesh,
        out_type=[jax.ShapeDtypeStruct((_NC, _N, d2), jnp.float32),
                  jax.ShapeDtypeStruct((_NC, _N, 16), jnp.float32)],
        scratch_types=[
            pltpu.VMEM((_SUB, _CH), jnp.int32),
            pltpu.VMEM((_SUB, _CH), jnp.int32),
            pltpu.VMEM((_SUB, _CH, d2), jnp.float32),
            pltpu.VMEM((_CH, 16), jnp.float32),
            pltpu.VMEM_SHARED((_N_PAD, d2), jnp.float32),
            pltpu.VMEM_SHARED((_N_PAD, 16), jnp.float32),
            pltpu.SemaphoreType.DMA,
        ],
        compiler_params=pltpu.CompilerParams(use_tc_tiling_on_sc=False),
        name=f"sage_segsum_d{d2}",
    )(functools.partial(_agg_body, d2=d2))


_agg64 = _make_agg(_D_IN // _NC)
_agg32 = _make_agg(_N_CLS // _NC)


def _layer_tc_body(p_ref, c_ref, w0_ref, b0_ref, w1_ref, b1_ref, z_ref):
    ssum = jnp.concatenate([p_ref[0], p_ref[1]], axis=-1)
    cntv = c_ref[0, :, 0:1]
    mean = ssum / jnp.maximum(cntv, 1.0)
    mask = (cntv > 0.0).astype(jnp.float32)
    h = jnp.dot(mean, w0_ref[...], preferred_element_type=jnp.float32)
    h = jnp.maximum(h + b0_ref[...] * mask, 0.0)
    z_ref[...] = (jnp.dot(h, w1_ref[...], preferred_element_type=jnp.float32)
                  + b1_ref[...])


def _mean_tc_body(p_ref, c_ref, o_ref):
    ssum = jnp.concatenate([p_ref[0], p_ref[1]], axis=-1)
    cntv = c_ref[0, :, 0:1]
    o_ref[...] = ssum / jnp.maximum(cntv, 1.0)


_BM = 1000


def _layer_tc(p, c, w0, b0, w1, b1):
    return pl.pallas_call(
        _layer_tc_body,
        grid=(_N // _BM,),
        in_specs=[
            pl.BlockSpec((_NC, _BM, _D_IN // _NC), lambda i: (0, i, 0)),
            pl.BlockSpec((_NC, _BM, 16), lambda i: (0, i, 0)),
            pl.BlockSpec((_D_IN, _D_H), lambda i: (0, 0)),
            pl.BlockSpec((1, _D_H), lambda i: (0, 0)),
            pl.BlockSpec((_D_H, _N_CLS), lambda i: (0, 0)),
            pl.BlockSpec((1, _N_CLS), lambda i: (0, 0)),
        ],
        out_specs=pl.BlockSpec((_BM, _N_CLS), lambda i: (i, 0)),
        out_shape=jax.ShapeDtypeStruct((_N, _N_CLS), jnp.float32),
        name="sage_dense",
    )(p, c, w0, b0, w1, b1)


def _mean_tc(p, c):
    return pl.pallas_call(
        _mean_tc_body,
        grid=(_N // _BM,),
        in_specs=[
            pl.BlockSpec((_NC, _BM, _N_CLS // _NC), lambda i: (0, i, 0)),
            pl.BlockSpec((_NC, _BM, 16), lambda i: (0, i, 0)),
        ],
        out_specs=pl.BlockSpec((_BM, _N_CLS), lambda i: (i, 0)),
        out_shape=jax.ShapeDtypeStruct((_N, _N_CLS), jnp.float32),
        name="sage_mean",
    )(p, c)


def _pad_edges(ei):
    pad = _E_PAD - _E
    ar = jnp.arange(pad, dtype=jnp.int32)
    src = jnp.concatenate([ei[0].astype(jnp.int32), (ar * 131) % _N])
    dst = jnp.concatenate([ei[1].astype(jnp.int32), _N + ar % _PAD_ROWS])
    return (src.reshape(_E_PAD // _CH, _CH), dst.reshape(_E_PAD // _CH, _CH))


def _split_cols(a):
    d2 = a.shape[1] // _NC
    return jnp.stack([a[:, :d2], a[:, d2:]])


def kernel(x, edge_index0, edge_index1, W0, b0, W1, b1):
    src0, dst0 = _pad_edges(edge_index0)
    src1, dst1 = _pad_edges(edge_index1)
    z64 = jnp.zeros((_N_PAD, _D_IN // _NC), jnp.float32)
    z32 = jnp.zeros((_N_PAD, _N_CLS // _NC), jnp.float32)
    z16 = jnp.zeros((_N_PAD, 16), jnp.float32)
    ones16 = jnp.ones((_CH, 16), jnp.float32)

    p0, c0 = _agg64(_split_cols(x), src0, dst0, z64, z16, ones16)
    z = _layer_tc(p0, c0, W0, b0.reshape(1, -1), W1, b1.reshape(1, -1))
    p1, c1 = _agg32(_split_cols(z), src1, dst1, z32, z16, ones16)
    return _mean_tc(p1, c1)


# speedup vs baseline: 11.6418x; 1.3857x over previous
import functools

import jax
import jax.numpy as jnp
from jax import lax
from jax.experimental import pallas as pl
from jax.experimental.pallas import tpu as pltpu
from jax.experimental.pallas import tpu_sc as plsc

_N = 10000
_D_IN = 128
_D_H = 256
_N_CLS = 64
_E = 320000

_NC = 2
_NS = 16
_CH = 128
_SUB = 4
_EPT = 20480
_E_PAD = _NS * _EPT
_ITERS = _EPT // (_CH * _SUB)
_N_PAD = 10240
_PAD_ROWS = _N_PAD - _N
_OPT = 624


def _agg_body(table, src2d, dst2d, zrows, z16, ones16, out_sum, out_cnt,
              src_v, dst_v, rows_v, ones_v, acc, cnt, sem0, sem1, *, d2):
    c = lax.axis_index("c")
    s = lax.axis_index("s")
    sems = (sem0, sem1)

    rpt = _N_PAD // _NS
    r0 = s * rpt
    pltpu.sync_copy(zrows.at[pl.ds(r0, rpt)], acc.at[pl.ds(r0, rpt)])
    pltpu.sync_copy(z16.at[pl.ds(r0, rpt)], cnt.at[pl.ds(r0, rpt)])
    pltpu.sync_copy(ones16, ones_v)
    plsc.subcore_barrier()

    row0 = s * (_EPT // _CH)
    tbl = table.at[c]
    half = _ITERS // 2

    def _fetch(k, blk):
        rb = row0 + blk * _SUB
        pltpu.sync_copy(src2d.at[pl.ds(rb, _SUB)], src_v.at[k])
        pltpu.sync_copy(dst2d.at[pl.ds(rb, _SUB)], dst_v.at[k])
        for j in range(_SUB):
            pltpu.async_copy(tbl.at[src_v.at[k, j]], rows_v.at[k, j], sems[k])

    for k in range(2):
        _fetch(k, k)

    def body(io, carry):
        for k in range(2):
            i = io * 2 + k
            for j in range(_SUB):
                pltpu.make_async_copy(tbl.at[src_v.at[k, j]],
                                      rows_v.at[k, j], sems[k]).wait()
            for j in range(_SUB):
                pltpu.sync_copy(rows_v.at[k, j], acc.at[dst_v.at[k, j]],
                                add=True)
            docnt = jnp.where(c == 0, i < half, i >= half)

            @pl.when(docnt)
            def _():
                for j in range(_SUB):
                    pltpu.sync_copy(ones_v, cnt.at[dst_v.at[k, j]], add=True)

            @pl.when(i + 2 < _ITERS)
            def _():
                _fetch(k, i + 2)
        return carry

    lax.fori_loop(0, _ITERS // 2, body, 0)
    plsc.subcore_barrier()

    o0 = s * _OPT
    rem0 = (_NS - 1) * _OPT
    rem = _N - rem0

    @pl.when(s < _NS - 1)
    def _():
        pltpu.sync_copy(acc.at[pl.ds(o0, _OPT)], out_sum.at[c, pl.ds(o0, _OPT)])
        pltpu.sync_copy(cnt.at[pl.ds(o0, _OPT)], out_cnt.at[c, pl.ds(o0, _OPT)])

    @pl.when(s == _NS - 1)
    def _():
        pltpu.sync_copy(acc.at[pl.ds(rem0, rem)], out_sum.at[c, pl.ds(rem0, rem)])
        pltpu.sync_copy(cnt.at[pl.ds(rem0, rem)], out_cnt.at[c, pl.ds(rem0, rem)])


def _make_agg(d2):
    mesh = plsc.VectorSubcoreMesh(core_axis_name="c", subcore_axis_name="s",
                                  num_cores=_NC, num_subcores=_NS)
    return functools.partial(
        pl.kernel,
        mesh=mesh,
        out_type=[jax.ShapeDtypeStruct((_NC, _N, d2), jnp.float32),
                  jax.ShapeDtypeStruct((_NC, _N, 16), jnp.float32)],
        scratch_types=[
            pltpu.VMEM((2, _SUB, _CH), jnp.int32),
            pltpu.VMEM((2, _SUB, _CH), jnp.int32),
            pltpu.VMEM((2, _SUB, _CH, d2), jnp.float32),
            pltpu.VMEM((_CH, 16), jnp.float32),
            pltpu.VMEM_SHARED((_N_PAD, d2), jnp.float32),
            pltpu.VMEM_SHARED((_N_PAD, 16), jnp.float32),
            pltpu.SemaphoreType.DMA,
            pltpu.SemaphoreType.DMA,
        ],
        compiler_params=pltpu.CompilerParams(use_tc_tiling_on_sc=False),
        name=f"sage_segsum_d{d2}",
    )(functools.partial(_agg_body, d2=d2))


_agg64 = _make_agg(_D_IN // _NC)
_agg32 = _make_agg(_N_CLS // _NC)


def _layer_tc_body(p_ref, c_ref, w0_ref, b0_ref, w1_ref, b1_ref, z_ref):
    ssum = jnp.concatenate([p_ref[0], p_ref[1]], axis=-1)
    cntv = c_ref[0, :, 0:1] + c_ref[1, :, 0:1]
    mean = ssum / jnp.maximum(cntv, 1.0)
    mask = (cntv > 0.0).astype(jnp.float32)
    h = jnp.dot(mean, w0_ref[...], preferred_element_type=jnp.float32)
    h = jnp.maximum(h + b0_ref[...] * mask, 0.0)
    z_ref[...] = (jnp.dot(h, w1_ref[...], preferred_element_type=jnp.float32)
                  + b1_ref[...])


def _mean_tc_body(p_ref, c_ref, o_ref):
    ssum = jnp.concatenate([p_ref[0], p_ref[1]], axis=-1)
    cntv = c_ref[0, :, 0:1] + c_ref[1, :, 0:1]
    o_ref[...] = ssum / jnp.maximum(cntv, 1.0)


_BM = 1000


def _layer_tc(p, c, w0, b0, w1, b1):
    return pl.pallas_call(
        _layer_tc_body,
        grid=(_N // _BM,),
        in_specs=[
            pl.BlockSpec((_NC, _BM, _D_IN // _NC), lambda i: (0, i, 0)),
            pl.BlockSpec((_NC, _BM, 16), lambda i: (0, i, 0)),
            pl.BlockSpec((_D_IN, _D_H), lambda i: (0, 0)),
            pl.BlockSpec((1, _D_H), lambda i: (0, 0)),
            pl.BlockSpec((_D_H, _N_CLS), lambda i: (0, 0)),
            pl.BlockSpec((1, _N_CLS), lambda i: (0, 0)),
        ],
        out_specs=pl.BlockSpec((_BM, _N_CLS), lambda i: (i, 0)),
        out_shape=jax.ShapeDtypeStruct((_N, _N_CLS), jnp.float32),
        name="sage_dense",
    )(p, c, w0, b0, w1, b1)


def _mean_tc(p, c):
    return pl.pallas_call(
        _mean_tc_body,
        grid=(_N // _BM,),
        in_specs=[
            pl.BlockSpec((_NC, _BM, _N_CLS // _NC), lambda i: (0, i, 0)),
            pl.BlockSpec((_NC, _BM, 16), lambda i: (0, i, 0)),
        ],
        out_specs=pl.BlockSpec((_BM, _N_CLS), lambda i: (i, 0)),
        out_shape=jax.ShapeDtypeStruct((_N, _N_CLS), jnp.float32),
        name="sage_mean",
    )(p, c)


def _pad_edges(ei):
    pad = _E_PAD - _E
    ar = jnp.arange(pad, dtype=jnp.int32)
    src = jnp.concatenate([ei[0].astype(jnp.int32), (ar * 131) % _N])
    dst = jnp.concatenate([ei[1].astype(jnp.int32), _N + ar % _PAD_ROWS])
    return (src.reshape(_E_PAD // _CH, _CH), dst.reshape(_E_PAD // _CH, _CH))


def _split_cols(a):
    d2 = a.shape[1] // _NC
    return jnp.stack([a[:, :d2], a[:, d2:]])


def kernel(x, edge_index0, edge_index1, W0, b0, W1, b1):
    src0, dst0 = _pad_edges(edge_index0)
    src1, dst1 = _pad_edges(edge_index1)
    z64 = jnp.zeros((_N_PAD, _D_IN // _NC), jnp.float32)
    z32 = jnp.zeros((_N_PAD, _N_CLS // _NC), jnp.float32)
    z16 = jnp.zeros((_N_PAD, 16), jnp.float32)
    ones16 = jnp.ones((_CH, 16), jnp.float32)

    p0, c0 = _agg64(_split_cols(x), src0, dst0, z64, z16, ones16)
    z = _layer_tc(p0, c0, W0, b0.reshape(1, -1), W1, b1.reshape(1, -1))
    p1, c1 = _agg32(_split_cols(z), src1, dst1, z32, z16, ones16)
    return _mean_tc(p1, c1)
